# baseline (speedup 1.0000x reference)
import jax
import jax.numpy as jnp
from jax import lax
from jax.experimental import pallas as pl
from jax.experimental.pallas import tpu as pltpu


def _conv2d(x, w, b=None, stride=1, padding=0):
    out = jax.lax.conv_general_dilated(
        x, w, (stride, stride), ((padding, padding), (padding, padding)),
        dimension_numbers=('NCHW', 'OIHW', 'NCHW'))
    if b is not None:
        out = out + b[None, :, None, None]
    return out


def _convT2d(x, w, b=None, stride=2, padding=1):
    k = w.shape[2]
    wf = jnp.transpose(w, (1, 0, 2, 3))[:, :, ::-1, ::-1]
    pad = k - 1 - padding
    out = jax.lax.conv_general_dilated(
        x, wf, (1, 1), ((pad, pad), (pad, pad)),
        lhs_dilation=(stride, stride),
        dimension_numbers=('NCHW', 'OIHW', 'NCHW'))
    if b is not None:
        out = out + b[None, :, None, None]
    return out


def _res_stack(x, w1, w2):
    for _ in range(2):
        h = _conv2d(jax.nn.relu(x), w1, None, 1, 1)
        h = _conv2d(jax.nn.relu(h), w2, None, 1, 0)
        x = x + h
    return jax.nn.relu(x)


_K = 8192
_D = 32
_NP = 3136
_PP = 3200
_BR = 128
_SPB = _PP // _BR


def _vq_body(fnorm_ref, cnorm_ref, z_ref, cb_ref, cbt_ref, q_ref, counts_ref):
    i = pl.program_id(0)
    zb = z_ref[0]
    cb = cb_ref[...]
    mm = lax.dot_general(cb, zb, (((1,), (0,)), ((), ())),
                         preferred_element_type=jnp.float32)
    dist = (fnorm_ref[0] + cnorm_ref[...]) - 2.0 * mm
    m = jnp.min(dist, axis=0, keepdims=True)
    iota = lax.broadcasted_iota(jnp.int32, (_K, _BR), 0)
    idx = jnp.min(jnp.where(dist == m, iota, _K), axis=0, keepdims=True)
    onehot = (iota == idx).astype(jnp.float32)
    q_ref[0] = lax.dot_general(cbt_ref[...], onehot, (((1,), (0,)), ((), ())),
                               preferred_element_type=jnp.float32)

    @pl.when(i == 0)
    def _init():
        counts_ref[...] = jnp.zeros_like(counts_ref)

    pix = (i % _SPB) * _BR + lax.broadcasted_iota(jnp.int32, (1, _BR), 1)
    valid = (pix < _NP).astype(jnp.float32)
    counts_ref[...] += jnp.sum(onehot * valid, axis=1, keepdims=True)


def _vq(fnorm, cnorm, z3, cb, cbt):
    grid = (z3.shape[0] * z3.shape[2]) // _BR
    return pl.pallas_call(
        _vq_body,
        grid=(grid,),
        in_specs=[
            pl.BlockSpec((1, 1, _BR), lambda i: (i // _SPB, 0, i % _SPB)),
            pl.BlockSpec((_K, 1), lambda i: (0, 0)),
            pl.BlockSpec((1, _D, _BR), lambda i: (i // _SPB, 0, i % _SPB)),
            pl.BlockSpec((_K, _D), lambda i: (0, 0)),
            pl.BlockSpec((_D, _K), lambda i: (0, 0)),
        ],
        out_specs=[
            pl.BlockSpec((1, _D, _BR), lambda i: (i // _SPB, 0, i % _SPB)),
            pl.BlockSpec((_K, 1), lambda i: (0, 0)),
        ],
        out_shape=[
            jax.ShapeDtypeStruct(z3.shape, jnp.float32),
            jax.ShapeDtypeStruct((_K, 1), jnp.float32),
        ],
    )(fnorm, cnorm, z3, cb, cbt)


def kernel(x, e_conv1_w, e_conv1_b, e_conv2_w, e_conv2_b, e_conv3_w, e_conv3_b,
           e_res_w1, e_res_w2, e_conv4_w, e_conv4_b, codebook,
           d_convT1_w, d_convT1_b, d_res_w1, d_res_w2,
           d_convT2_w, d_convT2_b, d_convT3_w, d_convT3_b):
    h = jax.nn.relu(_conv2d(x, e_conv1_w, e_conv1_b, 2, 1))
    h = jax.nn.relu(_conv2d(h, e_conv2_w, e_conv2_b, 2, 1))
    h = _conv2d(h, e_conv3_w, e_conv3_b, 1, 1)
    h = _res_stack(h, e_res_w1, e_res_w2)
    z = _conv2d(h, e_conv4_w, e_conv4_b, 1, 1)
    b, c, hh, ww = z.shape
    z3 = jnp.pad(z.reshape(b, c, hh * ww), ((0, 0), (0, 0), (0, _PP - _NP)))
    n = b * hh * ww
    fnorm = jnp.pad(jnp.sum(z ** 2, axis=1).reshape(b, 1, hh * ww),
                    ((0, 0), (0, 0), (0, _PP - _NP)))
    cnorm = jnp.sum(codebook ** 2, axis=1)[:, None]
    cbt = jnp.transpose(codebook)
    q3, counts = _vq(fnorm, cnorm, z3, codebook, cbt)
    quantized = q3[:, :, :_NP].reshape(z.shape)
    e_latent_loss = jnp.mean((jax.lax.stop_gradient(quantized) - z) ** 2)
    q_latent_loss = jnp.mean((quantized - jax.lax.stop_gradient(z)) ** 2)
    vq_loss = q_latent_loss + 0.25 * e_latent_loss
    quantized_out = z + jax.lax.stop_gradient(quantized - z)
    avg_probs = counts[:, 0] / n
    perplexity = jnp.exp(-jnp.sum(avg_probs * jnp.log(avg_probs + 1e-10)))
    h = _convT2d(z, d_convT1_w, d_convT1_b, 1, 1)
    h = _res_stack(h, d_res_w1, d_res_w2)
    h = jax.nn.relu(_convT2d(h, d_convT2_w, d_convT2_b, 2, 1))
    x_recon = _convT2d(h, d_convT3_w, d_convT3_b, 2, 1)
    return (x_recon, vq_loss, perplexity, quantized_out)


# speedup vs baseline: 1.2579x; 1.2579x over previous
import functools

import jax
import jax.numpy as jnp
from jax import lax
from jax.experimental import pallas as pl
from jax.experimental.pallas import tpu as pltpu
from jax.experimental.pallas import tpu_sc as plsc


def _conv2d(x, w, b=None, stride=1, padding=0):
    out = jax.lax.conv_general_dilated(
        x, w, (stride, stride), ((padding, padding), (padding, padding)),
        dimension_numbers=('NCHW', 'OIHW', 'NCHW'))
    if b is not None:
        out = out + b[None, :, None, None]
    return out


def _convT2d(x, w, b=None, stride=2, padding=1):
    k = w.shape[2]
    wf = jnp.transpose(w, (1, 0, 2, 3))[:, :, ::-1, ::-1]
    pad = k - 1 - padding
    out = jax.lax.conv_general_dilated(
        x, wf, (1, 1), ((pad, pad), (pad, pad)),
        lhs_dilation=(stride, stride),
        dimension_numbers=('NCHW', 'OIHW', 'NCHW'))
    if b is not None:
        out = out + b[None, :, None, None]
    return out


def _res_stack(x, w1, w2):
    for _ in range(2):
        h = _conv2d(jax.nn.relu(x), w1, None, 1, 1)
        h = _conv2d(jax.nn.relu(h), w2, None, 1, 0)
        x = x + h
    return jax.nn.relu(x)


_K = 8192
_D = 32
_NP = 3136
_PP = 3200
_BR = 128
_SPB = _PP // _BR


def _vq_body(fnorm_ref, cnorm_ref, z_ref, cb_ref, idx_ref, counts_ref):
    i = pl.program_id(0)
    flat = z_ref[0].T
    cb = cb_ref[...]
    mm = lax.dot_general(flat, cb, (((1,), (1,)), ((), ())),
                         preferred_element_type=jnp.float32)
    dist = (fnorm_ref[...] + cnorm_ref[...]) - 2.0 * mm
    m = jnp.min(dist, axis=1, keepdims=True)
    iota = lax.broadcasted_iota(jnp.int32, (_BR, _K), 1)
    idx = jnp.min(jnp.where(dist == m, iota, _K), axis=1)
    idx_ref[0, 0, :] = idx
    onehot = (iota == idx[:, None]).astype(jnp.float32)

    @pl.when(i == 0)
    def _init():
        counts_ref[...] = jnp.zeros_like(counts_ref)

    pix = (i % _SPB) * _BR + lax.broadcasted_iota(jnp.int32, (_BR, 1), 0)
    valid = (pix < _NP).astype(jnp.float32)
    counts_ref[...] += jnp.sum(onehot * valid, axis=0, keepdims=True)


def _vq_idx(fnorm, cnorm, z3, cb):
    grid = (z3.shape[0] * z3.shape[2]) // _BR
    return pl.pallas_call(
        _vq_body,
        grid=(grid,),
        in_specs=[
            pl.BlockSpec((_BR, 1), lambda i: (i, 0)),
            pl.BlockSpec((1, _K), lambda i: (0, 0)),
            pl.BlockSpec((1, _D, _BR), lambda i: (i // _SPB, 0, i % _SPB)),
            pl.BlockSpec((_K, _D), lambda i: (0, 0)),
        ],
        out_specs=[
            pl.BlockSpec((1, 1, _BR), lambda i: (i, 0, 0)),
            pl.BlockSpec((1, _K), lambda i: (0, 0)),
        ],
        out_shape=[
            jax.ShapeDtypeStruct((grid, 1, _BR), jnp.int32),
            jax.ShapeDtypeStruct((1, _K), jnp.float32),
        ],
    )(fnorm, cnorm, z3, cb)


_NIDX = 2 * _PP


def _sc_gather(idx_pad, cbt):
    mesh = plsc.VectorSubcoreMesh(core_axis_name="c", subcore_axis_name="s")

    @functools.partial(
        pl.kernel, mesh=mesh,
        compiler_params=pltpu.CompilerParams(needs_layout_passes=False),
        out_type=jax.ShapeDtypeStruct((2, _D, _PP), jnp.float32),
        scratch_types=[
            pltpu.VMEM((_NIDX,), jnp.int32),
            pltpu.VMEM((_K,), jnp.float32),
            pltpu.VMEM((_NIDX,), jnp.float32),
        ],
    )
    def k(idx_hbm, cbt_hbm, q_hbm, idx_v, tab_v, out_v):
        w = lax.axis_index("s") * 2 + lax.axis_index("c")
        pltpu.sync_copy(idx_hbm, idx_v)
        pltpu.sync_copy(cbt_hbm.at[w], tab_v)

        def body(j, carry):
            iv = idx_v[pl.ds(j * 16, 16)]
            out_v[pl.ds(j * 16, 16)] = plsc.load_gather(tab_v, [iv])
            return carry

        lax.fori_loop(0, _NIDX // 16, body, 0, unroll=8)
        pltpu.sync_copy(out_v.at[pl.ds(0, _PP)], q_hbm.at[0, w])
        pltpu.sync_copy(out_v.at[pl.ds(_PP, _PP)], q_hbm.at[1, w])

    return k(idx_pad, cbt)


def kernel(x, e_conv1_w, e_conv1_b, e_conv2_w, e_conv2_b, e_conv3_w, e_conv3_b,
           e_res_w1, e_res_w2, e_conv4_w, e_conv4_b, codebook,
           d_convT1_w, d_convT1_b, d_res_w1, d_res_w2,
           d_convT2_w, d_convT2_b, d_convT3_w, d_convT3_b):
    h = jax.nn.relu(_conv2d(x, e_conv1_w, e_conv1_b, 2, 1))
    h = jax.nn.relu(_conv2d(h, e_conv2_w, e_conv2_b, 2, 1))
    h = _conv2d(h, e_conv3_w, e_conv3_b, 1, 1)
    h = _res_stack(h, e_res_w1, e_res_w2)
    z = _conv2d(h, e_conv4_w, e_conv4_b, 1, 1)
    b, c, hh, ww = z.shape
    n = b * hh * ww
    z3 = jnp.pad(z.reshape(b, c, hh * ww), ((0, 0), (0, 0), (0, _PP - _NP)))
    fnorm = jnp.pad(jnp.sum(z ** 2, axis=1).reshape(b, hh * ww),
                    ((0, 0), (0, _PP - _NP))).reshape(b * _PP, 1)
    cnorm = jnp.sum(codebook ** 2, axis=1)[None, :]
    idx3, counts = _vq_idx(fnorm, cnorm, z3, codebook)
    cbt = jnp.transpose(codebook)
    q3 = _sc_gather(idx3.reshape(_NIDX), cbt)
    quantized = q3[:, :, :_NP].reshape(z.shape)
    e_latent_loss = jnp.mean((jax.lax.stop_gradient(quantized) - z) ** 2)
    q_latent_loss = jnp.mean((quantized - jax.lax.stop_gradient(z)) ** 2)
    vq_loss = q_latent_loss + 0.25 * e_latent_loss
    quantized_out = z + jax.lax.stop_gradient(quantized - z)
    avg_probs = counts[0] / n
    perplexity = jnp.exp(-jnp.sum(avg_probs * jnp.log(avg_probs + 1e-10)))
    h = _convT2d(z, d_convT1_w, d_convT1_b, 1, 1)
    h = _res_stack(h, d_res_w1, d_res_w2)
    h = jax.nn.relu(_convT2d(h, d_convT2_w, d_convT2_b, 2, 1))
    x_recon = _convT2d(h, d_convT3_w, d_convT3_b, 2, 1)
    return (x_recon, vq_loss, perplexity, quantized_out)


# speedup vs baseline: 1.2699x; 1.0095x over previous
import functools

import jax
import jax.numpy as jnp
from jax import lax
from jax.experimental import pallas as pl
from jax.experimental.pallas import tpu as pltpu
from jax.experimental.pallas import tpu_sc as plsc


def _conv2d(x, w, b=None, stride=1, padding=0):
    out = jax.lax.conv_general_dilated(
        x, w, (stride, stride), ((padding, padding), (padding, padding)),
        dimension_numbers=('NCHW', 'OIHW', 'NCHW'))
    if b is not None:
        out = out + b[None, :, None, None]
    return out


def _convT2d(x, w, b=None, stride=2, padding=1):
    k = w.shape[2]
    wf = jnp.transpose(w, (1, 0, 2, 3))[:, :, ::-1, ::-1]
    pad = k - 1 - padding
    out = jax.lax.conv_general_dilated(
        x, wf, (1, 1), ((pad, pad), (pad, pad)),
        lhs_dilation=(stride, stride),
        dimension_numbers=('NCHW', 'OIHW', 'NCHW'))
    if b is not None:
        out = out + b[None, :, None, None]
    return out


def _res_stack(x, w1, w2):
    for _ in range(2):
        h = _conv2d(jax.nn.relu(x), w1, None, 1, 1)
        h = _conv2d(jax.nn.relu(h), w2, None, 1, 0)
        x = x + h
    return jax.nn.relu(x)


_K = 8192
_D = 32
_NP = 3136
_PP = 3200
_BR = 128
_SPB = _PP // _BR


def _vq_body(fnorm_ref, cnorm_ref, z_ref, cb_ref, idx_ref, counts_ref):
    i = pl.program_id(0)
    flat = z_ref[0].T
    cb = cb_ref[...]
    mm = lax.dot_general(flat, cb, (((1,), (1,)), ((), ())),
                         preferred_element_type=jnp.float32)
    dist = (fnorm_ref[...] + cnorm_ref[...]) - 2.0 * mm
    m = jnp.min(dist, axis=1, keepdims=True)
    iota = lax.broadcasted_iota(jnp.int32, (_BR, _K), 1)
    idx = jnp.minimum(jnp.min(jnp.where(dist == m, iota, _K), axis=1), _K - 1)
    idx_ref[0, 0, :] = idx
    onehot = (iota == idx[:, None]).astype(jnp.float32)

    @pl.when(i == 0)
    def _init():
        counts_ref[...] = jnp.zeros_like(counts_ref)

    pix = (i % _SPB) * _BR + lax.broadcasted_iota(jnp.int32, (_BR, 1), 0)
    valid = (pix < _NP).astype(jnp.float32)
    counts_ref[...] += jnp.sum(onehot * valid, axis=0, keepdims=True)


def _vq_idx(fnorm, cnorm, z3, cb):
    grid = (z3.shape[0] * _PP) // _BR
    return pl.pallas_call(
        _vq_body,
        grid=(grid,),
        in_specs=[
            pl.BlockSpec((_BR, 1), lambda i: (i, 0)),
            pl.BlockSpec((1, _K), lambda i: (0, 0)),
            pl.BlockSpec((1, _D, _BR), lambda i: (i // _SPB, 0, i % _SPB)),
            pl.BlockSpec((_K, _D), lambda i: (0, 0)),
        ],
        out_specs=[
            pl.BlockSpec((1, 1, _BR), lambda i: (i, 0, 0)),
            pl.BlockSpec((1, _K), lambda i: (0, 0)),
        ],
        out_shape=[
            jax.ShapeDtypeStruct((grid, 1, _BR), jnp.int32),
            jax.ShapeDtypeStruct((1, _K), jnp.float32),
        ],
    )(fnorm, cnorm, z3, cb)


_NIDX = 2 * _PP


def _sc_gather(idx_pad, cbt):
    mesh = plsc.VectorSubcoreMesh(core_axis_name="c", subcore_axis_name="s")

    @functools.partial(
        pl.kernel, mesh=mesh,
        compiler_params=pltpu.CompilerParams(needs_layout_passes=False),
        out_type=jax.ShapeDtypeStruct((2 * _D * _NP,), jnp.float32),
        scratch_types=[
            pltpu.VMEM((_NIDX,), jnp.int32),
            pltpu.VMEM((_K,), jnp.float32),
            pltpu.VMEM((_NIDX,), jnp.float32),
        ],
    )
    def k(idx_hbm, cbt_hbm, q_hbm, idx_v, tab_v, out_v):
        w = lax.axis_index("s") * 2 + lax.axis_index("c")
        pltpu.sync_copy(idx_hbm, idx_v)
        pltpu.sync_copy(cbt_hbm.at[w], tab_v)

        def body(j, carry):
            iv = idx_v[pl.ds(j * 16, 16)]
            out_v[pl.ds(j * 16, 16)] = plsc.load_gather(tab_v, [iv])
            return carry

        lax.fori_loop(0, _NIDX // 16, body, 0, unroll=8)
        pltpu.sync_copy(out_v.at[pl.ds(0, _NP)], q_hbm.at[pl.ds(w * _NP, _NP)])
        pltpu.sync_copy(out_v.at[pl.ds(_PP, _NP)],
                        q_hbm.at[pl.ds((_D + w) * _NP, _NP)])

    return k(idx_pad, cbt)


def kernel(x, e_conv1_w, e_conv1_b, e_conv2_w, e_conv2_b, e_conv3_w, e_conv3_b,
           e_res_w1, e_res_w2, e_conv4_w, e_conv4_b, codebook,
           d_convT1_w, d_convT1_b, d_res_w1, d_res_w2,
           d_convT2_w, d_convT2_b, d_convT3_w, d_convT3_b):
    h = jax.nn.relu(_conv2d(x, e_conv1_w, e_conv1_b, 2, 1))
    h = jax.nn.relu(_conv2d(h, e_conv2_w, e_conv2_b, 2, 1))
    h = _conv2d(h, e_conv3_w, e_conv3_b, 1, 1)
    h = _res_stack(h, e_res_w1, e_res_w2)
    z = _conv2d(h, e_conv4_w, e_conv4_b, 1, 1)
    b, c, hh, ww = z.shape
    n = b * hh * ww
    z3 = z.reshape(b, c, hh * ww)
    fnorm = jnp.pad(jnp.sum(z ** 2, axis=1).reshape(b, hh * ww),
                    ((0, 0), (0, _PP - _NP))).reshape(b * _PP, 1)
    cnorm = jnp.sum(codebook ** 2, axis=1)[None, :]
    idx3, counts = _vq_idx(fnorm, cnorm, z3, codebook)
    cbt = jnp.transpose(codebook)
    q3 = _sc_gather(idx3.reshape(_NIDX), cbt)
    quantized = q3.reshape(z.shape)
    e_latent_loss = jnp.mean((jax.lax.stop_gradient(quantized) - z) ** 2)
    q_latent_loss = jnp.mean((quantized - jax.lax.stop_gradient(z)) ** 2)
    vq_loss = q_latent_loss + 0.25 * e_latent_loss
    quantized_out = z + jax.lax.stop_gradient(quantized - z)
    avg_probs = counts[0] / n
    perplexity = jnp.exp(-jnp.sum(avg_probs * jnp.log(avg_probs + 1e-10)))
    h = _convT2d(z, d_convT1_w, d_convT1_b, 1, 1)
    h = _res_stack(h, d_res_w1, d_res_w2)
    h = jax.nn.relu(_convT2d(h, d_convT2_w, d_convT2_b, 2, 1))
    x_recon = _convT2d(h, d_convT3_w, d_convT3_b, 2, 1)
    return (x_recon, vq_loss, perplexity, quantized_out)


# speedup vs baseline: 1.2748x; 1.0039x over previous
import functools

import jax
import jax.numpy as jnp
from jax import lax
from jax.experimental import pallas as pl
from jax.experimental.pallas import tpu as pltpu
from jax.experimental.pallas import tpu_sc as plsc


def _conv2d(x, w, b=None, stride=1, padding=0):
    out = jax.lax.conv_general_dilated(
        x, w, (stride, stride), ((padding, padding), (padding, padding)),
        dimension_numbers=('NCHW', 'OIHW', 'NCHW'))
    if b is not None:
        out = out + b[None, :, None, None]
    return out


def _convT2d(x, w, b=None, stride=2, padding=1):
    k = w.shape[2]
    wf = jnp.transpose(w, (1, 0, 2, 3))[:, :, ::-1, ::-1]
    pad = k - 1 - padding
    out = jax.lax.conv_general_dilated(
        x, wf, (1, 1), ((pad, pad), (pad, pad)),
        lhs_dilation=(stride, stride),
        dimension_numbers=('NCHW', 'OIHW', 'NCHW'))
    if b is not None:
        out = out + b[None, :, None, None]
    return out


def _res_stack(x, w1, w2):
    for _ in range(2):
        h = _conv2d(jax.nn.relu(x), w1, None, 1, 1)
        h = _conv2d(jax.nn.relu(h), w2, None, 1, 0)
        x = x + h
    return jax.nn.relu(x)


_K = 8192
_D = 32
_NP = 3136
_PP = 3200
_BR = 128
_SPB = _PP // _BR


def _vq_body(fnorm_ref, cnorm_ref, kiota_ref, z_ref, cb_ref, idx_ref):
    flat = z_ref[0].T
    cb = cb_ref[...]
    mm = lax.dot_general(flat, cb, (((1,), (1,)), ((), ())),
                         preferred_element_type=jnp.float32)
    dist = (fnorm_ref[...] + cnorm_ref[...]) - 2.0 * mm
    m = jnp.min(dist, axis=1, keepdims=True)
    idx = jnp.minimum(
        jnp.min(jnp.where(dist == m, kiota_ref[...], _K), axis=1), _K - 1)
    idx_ref[0, 0, :] = idx


def _vq_idx(fnorm, cnorm, kiota, z3, cb):
    grid = (z3.shape[0] * _PP) // _BR
    return pl.pallas_call(
        _vq_body,
        grid=(grid,),
        in_specs=[
            pl.BlockSpec((_BR, 1), lambda i: (i, 0)),
            pl.BlockSpec((1, _K), lambda i: (0, 0)),
            pl.BlockSpec((1, _K), lambda i: (0, 0)),
            pl.BlockSpec((1, _D, _BR), lambda i: (i // _SPB, 0, i % _SPB)),
            pl.BlockSpec((_K, _D), lambda i: (0, 0)),
        ],
        out_specs=pl.BlockSpec((1, 1, _BR), lambda i: (i, 0, 0)),
        out_shape=jax.ShapeDtypeStruct((grid, 1, _BR), jnp.int32),
    )(fnorm, cnorm, kiota, z3, cb)


_NIDX = 2 * _PP


def _sc_gather(idx_pad, cbt):
    mesh = plsc.VectorSubcoreMesh(core_axis_name="c", subcore_axis_name="s")

    @functools.partial(
        pl.kernel, mesh=mesh,
        compiler_params=pltpu.CompilerParams(needs_layout_passes=False),
        out_type=jax.ShapeDtypeStruct((2 * _D * _NP,), jnp.float32),
        scratch_types=[
            pltpu.VMEM((_NIDX,), jnp.int32),
            pltpu.VMEM((_K,), jnp.float32),
            pltpu.VMEM((_NIDX,), jnp.float32),
        ],
    )
    def k(idx_hbm, cbt_hbm, q_hbm, idx_v, tab_v, out_v):
        w = lax.axis_index("s") * 2 + lax.axis_index("c")
        pltpu.sync_copy(idx_hbm, idx_v)
        pltpu.sync_copy(cbt_hbm.at[w], tab_v)

        def body(j, carry):
            iv = idx_v[pl.ds(j * 16, 16)]
            out_v[pl.ds(j * 16, 16)] = plsc.load_gather(tab_v, [iv])
            return carry

        lax.fori_loop(0, _NIDX // 16, body, 0, unroll=8)
        pltpu.sync_copy(out_v.at[pl.ds(0, _NP)], q_hbm.at[pl.ds(w * _NP, _NP)])
        pltpu.sync_copy(out_v.at[pl.ds(_PP, _NP)],
                        q_hbm.at[pl.ds((_D + w) * _NP, _NP)])

    return k(idx_pad, cbt)


def kernel(x, e_conv1_w, e_conv1_b, e_conv2_w, e_conv2_b, e_conv3_w, e_conv3_b,
           e_res_w1, e_res_w2, e_conv4_w, e_conv4_b, codebook,
           d_convT1_w, d_convT1_b, d_res_w1, d_res_w2,
           d_convT2_w, d_convT2_b, d_convT3_w, d_convT3_b):
    h = jax.nn.relu(_conv2d(x, e_conv1_w, e_conv1_b, 2, 1))
    h = jax.nn.relu(_conv2d(h, e_conv2_w, e_conv2_b, 2, 1))
    h = _conv2d(h, e_conv3_w, e_conv3_b, 1, 1)
    h = _res_stack(h, e_res_w1, e_res_w2)
    z = _conv2d(h, e_conv4_w, e_conv4_b, 1, 1)
    b, c, hh, ww = z.shape
    n = b * hh * ww
    z3 = z.reshape(b, c, hh * ww)
    fnorm = jnp.pad(jnp.sum(z ** 2, axis=1).reshape(b, hh * ww),
                    ((0, 0), (0, _PP - _NP))).reshape(b * _PP, 1)
    cnorm = jnp.sum(codebook ** 2, axis=1)[None, :]
    kiota = jnp.arange(_K, dtype=jnp.int32)[None, :]
    idx3 = _vq_idx(fnorm, cnorm, kiota, z3, codebook)
    cbt = jnp.transpose(codebook)
    q3 = _sc_gather(idx3.reshape(_NIDX), cbt)
    quantized = q3.reshape(z.shape)
    idx = idx3.reshape(b, _PP)[:, :_NP].reshape(b * hh * ww)
    e_latent_loss = jnp.mean((jax.lax.stop_gradient(quantized) - z) ** 2)
    q_latent_loss = jnp.mean((quantized - jax.lax.stop_gradient(z)) ** 2)
    vq_loss = q_latent_loss + 0.25 * e_latent_loss
    quantized_out = z + jax.lax.stop_gradient(quantized - z)
    avg_probs = jnp.bincount(idx, length=codebook.shape[0]).astype(jnp.float32) / n
    perplexity = jnp.exp(-jnp.sum(avg_probs * jnp.log(avg_probs + 1e-10)))
    h = _convT2d(z, d_convT1_w, d_convT1_b, 1, 1)
    h = _res_stack(h, d_res_w1, d_res_w2)
    h = jax.nn.relu(_convT2d(h, d_convT2_w, d_convT2_b, 2, 1))
    x_recon = _convT2d(h, d_convT3_w, d_convT3_b, 2, 1)
    return (x_recon, vq_loss, perplexity, quantized_out)


# speedup vs baseline: 1.3141x; 1.0308x over previous
import functools

import jax
import jax.numpy as jnp
from jax import lax
from jax.experimental import pallas as pl
from jax.experimental.pallas import tpu as pltpu
from jax.experimental.pallas import tpu_sc as plsc


def _conv2d(x, w, b=None, stride=1, padding=0):
    out = jax.lax.conv_general_dilated(
        x, w, (stride, stride), ((padding, padding), (padding, padding)),
        dimension_numbers=('NCHW', 'OIHW', 'NCHW'))
    if b is not None:
        out = out + b[None, :, None, None]
    return out


def _convT2d(x, w, b=None, stride=2, padding=1):
    k = w.shape[2]
    wf = jnp.transpose(w, (1, 0, 2, 3))[:, :, ::-1, ::-1]
    pad = k - 1 - padding
    out = jax.lax.conv_general_dilated(
        x, wf, (1, 1), ((pad, pad), (pad, pad)),
        lhs_dilation=(stride, stride),
        dimension_numbers=('NCHW', 'OIHW', 'NCHW'))
    if b is not None:
        out = out + b[None, :, None, None]
    return out


def _res_stack(x, w1, w2):
    for _ in range(2):
        h = _conv2d(jax.nn.relu(x), w1, None, 1, 1)
        h = _conv2d(jax.nn.relu(h), w2, None, 1, 0)
        x = x + h
    return jax.nn.relu(x)


_K = 8192
_D = 32
_NP = 3136
_PP = 3328
_BR = 256
_SPB = _PP // _BR


def _vq_body(fnorm_ref, cnorm_ref, kiota_ref, z_ref, cb_ref, idx_ref):
    flat = z_ref[0].T
    cb = cb_ref[...]
    mm = lax.dot_general(flat, cb, (((1,), (1,)), ((), ())),
                         preferred_element_type=jnp.float32)
    dist = (fnorm_ref[...] + cnorm_ref[...]) - 2.0 * mm
    m = jnp.min(dist, axis=1, keepdims=True)
    idx = jnp.minimum(
        jnp.min(jnp.where(dist == m, kiota_ref[...], _K), axis=1), _K - 1)
    idx_ref[0, 0, :] = idx


def _vq_idx(fnorm, cnorm, kiota, z3, cb):
    grid = (z3.shape[0] * _PP) // _BR
    return pl.pallas_call(
        _vq_body,
        grid=(grid,),
        in_specs=[
            pl.BlockSpec((_BR, 1), lambda i: (i, 0)),
            pl.BlockSpec((1, _K), lambda i: (0, 0)),
            pl.BlockSpec((1, _K), lambda i: (0, 0)),
            pl.BlockSpec((1, _D, _BR), lambda i: (i // _SPB, 0, i % _SPB)),
            pl.BlockSpec((_K, _D), lambda i: (0, 0)),
        ],
        out_specs=pl.BlockSpec((1, 1, _BR), lambda i: (i, 0, 0)),
        out_shape=jax.ShapeDtypeStruct((grid, 1, _BR), jnp.int32),
    )(fnorm, cnorm, kiota, z3, cb)


_NIDX = 2 * _PP


def _sc_gather(idx_pad, cbt):
    mesh = plsc.VectorSubcoreMesh(core_axis_name="c", subcore_axis_name="s")

    @functools.partial(
        pl.kernel, mesh=mesh,
        compiler_params=pltpu.CompilerParams(needs_layout_passes=False),
        out_type=jax.ShapeDtypeStruct((2 * _D * _NP,), jnp.float32),
        scratch_types=[
            pltpu.VMEM((_NIDX,), jnp.int32),
            pltpu.VMEM((_K,), jnp.float32),
            pltpu.VMEM((_NIDX,), jnp.float32),
        ],
    )
    def k(idx_hbm, cbt_hbm, q_hbm, idx_v, tab_v, out_v):
        w = lax.axis_index("s") * 2 + lax.axis_index("c")
        pltpu.sync_copy(idx_hbm, idx_v)
        pltpu.sync_copy(cbt_hbm.at[w], tab_v)

        def body(j, carry):
            iv = idx_v[pl.ds(j * 16, 16)]
            out_v[pl.ds(j * 16, 16)] = plsc.load_gather(tab_v, [iv])
            return carry

        lax.fori_loop(0, _NIDX // 16, body, 0, unroll=8)
        pltpu.sync_copy(out_v.at[pl.ds(0, _NP)], q_hbm.at[pl.ds(w * _NP, _NP)])
        pltpu.sync_copy(out_v.at[pl.ds(_PP, _NP)],
                        q_hbm.at[pl.ds((_D + w) * _NP, _NP)])

    return k(idx_pad, cbt)


def kernel(x, e_conv1_w, e_conv1_b, e_conv2_w, e_conv2_b, e_conv3_w, e_conv3_b,
           e_res_w1, e_res_w2, e_conv4_w, e_conv4_b, codebook,
           d_convT1_w, d_convT1_b, d_res_w1, d_res_w2,
           d_convT2_w, d_convT2_b, d_convT3_w, d_convT3_b):
    h = jax.nn.relu(_conv2d(x, e_conv1_w, e_conv1_b, 2, 1))
    h = jax.nn.relu(_conv2d(h, e_conv2_w, e_conv2_b, 2, 1))
    h = _conv2d(h, e_conv3_w, e_conv3_b, 1, 1)
    h = _res_stack(h, e_res_w1, e_res_w2)
    z = _conv2d(h, e_conv4_w, e_conv4_b, 1, 1)
    b, c, hh, ww = z.shape
    n = b * hh * ww
    z3 = z.reshape(b, c, hh * ww)
    fnorm = jnp.pad(jnp.sum(z ** 2, axis=1).reshape(b, hh * ww),
                    ((0, 0), (0, _PP - _NP))).reshape(b * _PP, 1)
    cnorm = jnp.sum(codebook ** 2, axis=1)[None, :]
    kiota = jnp.arange(_K, dtype=jnp.int32)[None, :]
    idx3 = _vq_idx(fnorm, cnorm, kiota, z3, codebook)
    cbt = jnp.transpose(codebook)
    q3 = _sc_gather(idx3.reshape(_NIDX), cbt)
    quantized = q3.reshape(z.shape)
    idx = idx3.reshape(b, _PP)[:, :_NP].reshape(b * hh * ww)
    e_latent_loss = jnp.mean((jax.lax.stop_gradient(quantized) - z) ** 2)
    q_latent_loss = jnp.mean((quantized - jax.lax.stop_gradient(z)) ** 2)
    vq_loss = q_latent_loss + 0.25 * e_latent_loss
    quantized_out = z + jax.lax.stop_gradient(quantized - z)
    avg_probs = jnp.bincount(idx, length=codebook.shape[0]).astype(jnp.float32) / n
    perplexity = jnp.exp(-jnp.sum(avg_probs * jnp.log(avg_probs + 1e-10)))
    h = _convT2d(z, d_convT1_w, d_convT1_b, 1, 1)
    h = _res_stack(h, d_res_w1, d_res_w2)
    h = jax.nn.relu(_convT2d(h, d_convT2_w, d_convT2_b, 2, 1))
    x_recon = _convT2d(h, d_convT3_w, d_convT3_b, 2, 1)
    return (x_recon, vq_loss, perplexity, quantized_out)


# speedup vs baseline: 1.3448x; 1.0234x over previous
import functools

import jax
import jax.numpy as jnp
from jax import lax
from jax.experimental import pallas as pl
from jax.experimental.pallas import tpu as pltpu
from jax.experimental.pallas import tpu_sc as plsc


def _conv2d(x, w, b=None, stride=1, padding=0):
    out = jax.lax.conv_general_dilated(
        x, w, (stride, stride), ((padding, padding), (padding, padding)),
        dimension_numbers=('NCHW', 'OIHW', 'NCHW'))
    if b is not None:
        out = out + b[None, :, None, None]
    return out


def _convT2d(x, w, b=None, stride=2, padding=1):
    k = w.shape[2]
    wf = jnp.transpose(w, (1, 0, 2, 3))[:, :, ::-1, ::-1]
    pad = k - 1 - padding
    out = jax.lax.conv_general_dilated(
        x, wf, (1, 1), ((pad, pad), (pad, pad)),
        lhs_dilation=(stride, stride),
        dimension_numbers=('NCHW', 'OIHW', 'NCHW'))
    if b is not None:
        out = out + b[None, :, None, None]
    return out


def _res_stack(x, w1, w2):
    for _ in range(2):
        h = _conv2d(jax.nn.relu(x), w1, None, 1, 1)
        h = _conv2d(jax.nn.relu(h), w2, None, 1, 0)
        x = x + h
    return jax.nn.relu(x)


_K = 8192
_D = 32
_NP = 3136
_PP = 3200
_BR = 640
_SPB = _PP // _BR


def _vq_body(fnorm_ref, cnorm_ref, kiota_ref, z_ref, cb_ref, idx_ref):
    flat = z_ref[0].T
    cb = cb_ref[...]
    mm = lax.dot_general(flat, cb, (((1,), (1,)), ((), ())),
                         preferred_element_type=jnp.float32)
    dist = (fnorm_ref[...] + cnorm_ref[...]) - 2.0 * mm
    m = jnp.min(dist, axis=1, keepdims=True)
    idx = jnp.minimum(
        jnp.min(jnp.where(dist == m, kiota_ref[...], _K), axis=1), _K - 1)
    idx_ref[0, 0, :] = idx


def _vq_idx(fnorm, cnorm, kiota, z3, cb):
    grid = (z3.shape[0] * _PP) // _BR
    return pl.pallas_call(
        _vq_body,
        grid=(grid,),
        in_specs=[
            pl.BlockSpec((_BR, 1), lambda i: (i, 0)),
            pl.BlockSpec((1, _K), lambda i: (0, 0)),
            pl.BlockSpec((1, _K), lambda i: (0, 0)),
            pl.BlockSpec((1, _D, _BR), lambda i: (i // _SPB, 0, i % _SPB)),
            pl.BlockSpec((_K, _D), lambda i: (0, 0)),
        ],
        out_specs=pl.BlockSpec((1, 1, _BR), lambda i: (i, 0, 0)),
        out_shape=jax.ShapeDtypeStruct((grid, 1, _BR), jnp.int32),
    )(fnorm, cnorm, kiota, z3, cb)


_NIDX = 2 * _PP


def _sc_gather(idx_pad, cbt):
    mesh = plsc.VectorSubcoreMesh(core_axis_name="c", subcore_axis_name="s")

    @functools.partial(
        pl.kernel, mesh=mesh,
        compiler_params=pltpu.CompilerParams(needs_layout_passes=False),
        out_type=jax.ShapeDtypeStruct((2 * _D * _NP,), jnp.float32),
        scratch_types=[
            pltpu.VMEM((_NIDX,), jnp.int32),
            pltpu.VMEM((_K,), jnp.float32),
            pltpu.VMEM((_NIDX,), jnp.float32),
        ],
    )
    def k(idx_hbm, cbt_hbm, q_hbm, idx_v, tab_v, out_v):
        w = lax.axis_index("s") * 2 + lax.axis_index("c")
        pltpu.sync_copy(idx_hbm, idx_v)
        pltpu.sync_copy(cbt_hbm.at[w], tab_v)

        def body(j, carry):
            iv = idx_v[pl.ds(j * 16, 16)]
            out_v[pl.ds(j * 16, 16)] = plsc.load_gather(tab_v, [iv])
            return carry

        lax.fori_loop(0, _NIDX // 16, body, 0, unroll=8)
        pltpu.sync_copy(out_v.at[pl.ds(0, _NP)], q_hbm.at[pl.ds(w * _NP, _NP)])
        pltpu.sync_copy(out_v.at[pl.ds(_PP, _NP)],
                        q_hbm.at[pl.ds((_D + w) * _NP, _NP)])

    return k(idx_pad, cbt)


def kernel(x, e_conv1_w, e_conv1_b, e_conv2_w, e_conv2_b, e_conv3_w, e_conv3_b,
           e_res_w1, e_res_w2, e_conv4_w, e_conv4_b, codebook,
           d_convT1_w, d_convT1_b, d_res_w1, d_res_w2,
           d_convT2_w, d_convT2_b, d_convT3_w, d_convT3_b):
    h = jax.nn.relu(_conv2d(x, e_conv1_w, e_conv1_b, 2, 1))
    h = jax.nn.relu(_conv2d(h, e_conv2_w, e_conv2_b, 2, 1))
    h = _conv2d(h, e_conv3_w, e_conv3_b, 1, 1)
    h = _res_stack(h, e_res_w1, e_res_w2)
    z = _conv2d(h, e_conv4_w, e_conv4_b, 1, 1)
    b, c, hh, ww = z.shape
    n = b * hh * ww
    z3 = z.reshape(b, c, hh * ww)
    fnorm = jnp.pad(jnp.sum(z ** 2, axis=1).reshape(b, hh * ww),
                    ((0, 0), (0, _PP - _NP))).reshape(b * _PP, 1)
    cnorm = jnp.sum(codebook ** 2, axis=1)[None, :]
    kiota = jnp.arange(_K, dtype=jnp.int32)[None, :]
    idx3 = _vq_idx(fnorm, cnorm, kiota, z3, codebook)
    cbt = jnp.transpose(codebook)
    q3 = _sc_gather(idx3.reshape(_NIDX), cbt)
    quantized = q3.reshape(z.shape)
    idx = idx3.reshape(b, _PP)[:, :_NP].reshape(b * hh * ww)
    e_latent_loss = jnp.mean((jax.lax.stop_gradient(quantized) - z) ** 2)
    q_latent_loss = jnp.mean((quantized - jax.lax.stop_gradient(z)) ** 2)
    vq_loss = q_latent_loss + 0.25 * e_latent_loss
    quantized_out = z + jax.lax.stop_gradient(quantized - z)
    avg_probs = jnp.bincount(idx, length=codebook.shape[0]).astype(jnp.float32) / n
    perplexity = jnp.exp(-jnp.sum(avg_probs * jnp.log(avg_probs + 1e-10)))
    h = _convT2d(z, d_convT1_w, d_convT1_b, 1, 1)
    h = _res_stack(h, d_res_w1, d_res_w2)
    h = jax.nn.relu(_convT2d(h, d_convT2_w, d_convT2_b, 2, 1))
    x_recon = _convT2d(h, d_convT3_w, d_convT3_b, 2, 1)
    return (x_recon, vq_loss, perplexity, quantized_out)
